# baseline (speedup 1.0000x reference)
import functools

import jax
import jax.numpy as jnp
from jax import lax
from jax.experimental import pallas as pl
from jax.experimental.pallas import tpu as pltpu
from jax.experimental.pallas import tpu_sc as plsc

_NC = 2
_NS = 16
_NW = _NC * _NS
_L = 16
_C = 32
_NBUF = 4


def _sc_gather_add(x, idx, table):
    T, D = x.shape
    V = table.shape[0]
    R = T // _NW
    chunks = R // _C
    nk = chunks // _NBUF

    mesh = plsc.VectorSubcoreMesh(core_axis_name="c", subcore_axis_name="s")

    buf = lambda: pltpu.VMEM((_C, D), jnp.float32)

    @functools.partial(
        pl.kernel,
        out_type=jax.ShapeDtypeStruct((T, D), jnp.float32),
        mesh=mesh,
        scratch_types=[
            pltpu.VMEM((V, D), jnp.float32),
            pltpu.VMEM((R,), jnp.int32),
            buf(), buf(), buf(), buf(),
            pltpu.SemaphoreType.DMA((_NBUF,)),
            pltpu.SemaphoreType.DMA((_NBUF,)),
        ],
    )
    def body(x_hbm, idx_hbm, tab_hbm, out_hbm, tab_v, idx_all,
             in0, in1, in2, in3, sem_in, sem_out):
        ins = (in0, in1, in2, in3)
        wid = lax.axis_index("s") * _NC + lax.axis_index("c")
        rbase = wid * R

        pltpu.sync_copy(tab_hbm, tab_v)
        pltpu.sync_copy(idx_hbm.at[pl.ds(rbase, R)], idx_all)

        def issue(i, b):
            base = rbase + i * _C
            pltpu.async_copy(x_hbm.at[pl.ds(base, _C)], ins[b], sem_in.at[b])

        for b in range(_NBUF):
            issue(b, b)

        def k_body(k, carry):
            for b in range(_NBUF):
                i = k * _NBUF + b
                base = rbase + i * _C
                pltpu.make_async_copy(
                    x_hbm.at[pl.ds(base, _C)], ins[b], sem_in.at[b]).wait()

                def addgrp(g, c2, _b=b, _i=i):
                    pv = idx_all[pl.ds(_i * _C + g * _L, _L)]
                    for j in range(_L):
                        p = pv[j]
                        r = g * _L + j
                        for d in range(D // _L):
                            sl = pl.ds(d * _L, _L)
                            plsc.addupdate(ins[_b].at[r, sl], tab_v[p, sl])
                    return c2

                lax.fori_loop(0, _C // _L, addgrp, 0, unroll=False)
                pltpu.async_copy(ins[b], out_hbm.at[pl.ds(base, _C)],
                                 sem_out.at[b])

            for b in range(_NBUF):
                i = (k + 1) * _NBUF + b

                @pl.when(i < chunks)
                def _(b=b, i=i):
                    prev = rbase + (i - _NBUF) * _C
                    pltpu.make_async_copy(
                        ins[b], out_hbm.at[pl.ds(prev, _C)],
                        sem_out.at[b]).wait()
                    issue(i, b)

            return carry

        lax.fori_loop(0, nk, k_body, 0, unroll=False)

        for b in range(_NBUF):
            i = (nk - 1) * _NBUF + b
            base = rbase + i * _C
            pltpu.make_async_copy(
                ins[b], out_hbm.at[pl.ds(base, _C)], sem_out.at[b]).wait()

    return body(x, idx, table)


def kernel(inputs, inputs_positions, position_emb):
    B, N, D = inputs.shape
    T = B * N
    x = inputs.reshape(T, D)
    idx = inputs_positions.reshape(T).astype(jnp.int32)
    table = position_emb.reshape(position_emb.shape[1], D)
    out = _sc_gather_add(x, idx, table)
    return out.reshape(B, N, D)


# speedup vs baseline: 1.9367x; 1.9367x over previous
import functools

import jax
import jax.numpy as jnp
from jax import lax
from jax.experimental import pallas as pl
from jax.experimental.pallas import tpu as pltpu
from jax.experimental.pallas import tpu_sc as plsc

_NC = 2
_NS = 16
_NW = _NC * _NS
_L = 16
_C = 32
_NBUF = 4
_M_SC = 32768
_G = 1024


def _sc_gather_add(x, idx, table, sc_base, m_sc):
    T, D = x.shape
    R = m_sc // _NW
    chunks = R // _C
    nk = chunks // _NBUF

    mesh = plsc.VectorSubcoreMesh(core_axis_name="c", subcore_axis_name="s")

    buf = lambda: pltpu.VMEM((_C, D), jnp.float32)

    @functools.partial(
        pl.kernel,
        out_type=jax.ShapeDtypeStruct((m_sc, D), jnp.float32),
        mesh=mesh,
        scratch_types=[
            pltpu.VMEM((R,), jnp.int32),
            buf(), buf(), buf(), buf(),
            buf(), buf(), buf(), buf(),
            pltpu.SemaphoreType.DMA((_NBUF,)),
            pltpu.SemaphoreType.DMA((_NBUF,)),
            pltpu.SemaphoreType.DMA((_NBUF,)),
        ],
    )
    def body(x_hbm, idx_hbm, tab_hbm, out_hbm, idx_all,
             in0, in1, in2, in3, row0, row1, row2, row3,
             sem_in, sem_row, sem_out):
        ins = (in0, in1, in2, in3)
        rows = (row0, row1, row2, row3)
        wid = lax.axis_index("s") * _NC + lax.axis_index("c")
        obase = wid * R
        rbase = sc_base + obase

        pltpu.sync_copy(idx_hbm.at[pl.ds(rbase, R)], idx_all)

        def issue(i, b):
            pltpu.async_copy(x_hbm.at[pl.ds(rbase + i * _C, _C)],
                             ins[b], sem_in.at[b])
            pltpu.async_copy(tab_hbm.at[idx_all.at[pl.ds(i * _C, _C)]],
                             rows[b], sem_row.at[b])

        for b in range(_NBUF):
            issue(b, b)

        def k_body(k, carry):
            for b in range(_NBUF):
                i = k * _NBUF + b
                pltpu.make_async_copy(
                    x_hbm.at[pl.ds(rbase + i * _C, _C)],
                    ins[b], sem_in.at[b]).wait()
                pltpu.make_async_copy(
                    tab_hbm.at[idx_all.at[pl.ds(i * _C, _C)]],
                    rows[b], sem_row.at[b]).wait()

                def addrow(r, c2, _b=b):
                    for d in range(D // _L):
                        sl = pl.ds(d * _L, _L)
                        plsc.addupdate(ins[_b].at[r, sl], rows[_b][r, sl])
                    return c2

                lax.fori_loop(0, _C, addrow, 0, unroll=2)
                pltpu.async_copy(ins[b], out_hbm.at[pl.ds(obase + i * _C, _C)],
                                 sem_out.at[b])

            for b in range(_NBUF):
                i = (k + 1) * _NBUF + b

                @pl.when(i < chunks)
                def _(b=b, i=i):
                    prev = obase + (i - _NBUF) * _C
                    pltpu.make_async_copy(
                        ins[b], out_hbm.at[pl.ds(prev, _C)],
                        sem_out.at[b]).wait()
                    issue(i, b)

            return carry

        lax.fori_loop(0, nk, k_body, 0, unroll=False)

        for b in range(_NBUF):
            i = (nk - 1) * _NBUF + b
            pltpu.make_async_copy(
                ins[b], out_hbm.at[pl.ds(obase + i * _C, _C)],
                sem_out.at[b]).wait()

    return body(x, idx, table)


def _tc_body(x_ref, pos_ref, tab_ref, out_ref):
    pos = pos_ref[...]
    oh = (pos[:, None] == lax.broadcasted_iota(jnp.int32, (_G, 128), 1)
          ).astype(jnp.float32)
    gathered = jnp.dot(oh, tab_ref[...], preferred_element_type=jnp.float32)
    out_ref[...] = x_ref[...] + gathered


def _tc_gather_add(x, idx, tab_pad, m_tc):
    T, D = x.shape
    grid = (m_tc // _G,)
    return pl.pallas_call(
        _tc_body,
        grid=grid,
        in_specs=[
            pl.BlockSpec((_G, D), lambda i: (i, 0)),
            pl.BlockSpec((_G,), lambda i: (i,)),
            pl.BlockSpec((128, D), lambda i: (0, 0)),
        ],
        out_specs=pl.BlockSpec((_G, D), lambda i: (i, 0)),
        out_shape=jax.ShapeDtypeStruct((T, D), jnp.float32),
    )(x, idx, tab_pad)


def kernel(inputs, inputs_positions, position_emb):
    B, N, D = inputs.shape
    T = B * N
    x = inputs.reshape(T, D)
    idx = inputs_positions.reshape(T).astype(jnp.int32)
    table = position_emb.reshape(position_emb.shape[1], D)
    V = table.shape[0]

    m_sc = _M_SC
    m_tc = T - m_sc

    out_sc = _sc_gather_add(x, idx, table, m_tc, m_sc)
    tab_pad = jnp.pad(table, ((0, 128 - V), (0, 0)))
    out_tc = _tc_gather_add(x, idx, tab_pad, m_tc)
    out = lax.dynamic_update_slice(out_tc, out_sc, (m_tc, 0))
    return out.reshape(B, N, D)


# speedup vs baseline: 1.9531x; 1.0085x over previous
import functools

import jax
import jax.numpy as jnp
from jax import lax
from jax.experimental import pallas as pl
from jax.experimental.pallas import tpu as pltpu
from jax.experimental.pallas import tpu_sc as plsc

_NC = 2
_NS = 16
_NW = _NC * _NS
_L = 16
_C = 32
_NBUF = 4
_M_SC = 32768
_G = 1024


def _sc_gather_add(x, idx, table, sc_base, m_sc):
    T, D = x.shape
    R = m_sc // _NW
    chunks = R // _C
    nk = chunks // _NBUF

    mesh = plsc.VectorSubcoreMesh(core_axis_name="c", subcore_axis_name="s")

    buf = lambda: pltpu.VMEM((_C, D), jnp.float32)

    @functools.partial(
        pl.kernel,
        out_type=jax.ShapeDtypeStruct((T, D), jnp.float32),
        mesh=mesh,
        scratch_types=[
            pltpu.VMEM((R,), jnp.int32),
            buf(), buf(), buf(), buf(),
            buf(), buf(), buf(), buf(),
            pltpu.SemaphoreType.DMA((_NBUF,)),
            pltpu.SemaphoreType.DMA((_NBUF,)),
            pltpu.SemaphoreType.DMA((_NBUF,)),
        ],
    )
    def body(x_hbm, idx_hbm, tab_hbm, out_hbm, idx_all,
             in0, in1, in2, in3, row0, row1, row2, row3,
             sem_in, sem_row, sem_out):
        ins = (in0, in1, in2, in3)
        rows = (row0, row1, row2, row3)
        wid = lax.axis_index("s") * _NC + lax.axis_index("c")
        rbase = sc_base + wid * R
        obase = rbase

        pltpu.sync_copy(idx_hbm.at[pl.ds(rbase, R)], idx_all)

        def issue(i, b):
            pltpu.async_copy(x_hbm.at[pl.ds(rbase + i * _C, _C)],
                             ins[b], sem_in.at[b])
            pltpu.async_copy(tab_hbm.at[idx_all.at[pl.ds(i * _C, _C)]],
                             rows[b], sem_row.at[b])

        for b in range(_NBUF):
            issue(b, b)

        def k_body(k, carry):
            for b in range(_NBUF):
                i = k * _NBUF + b
                pltpu.make_async_copy(
                    x_hbm.at[pl.ds(rbase + i * _C, _C)],
                    ins[b], sem_in.at[b]).wait()
                pltpu.make_async_copy(
                    tab_hbm.at[idx_all.at[pl.ds(i * _C, _C)]],
                    rows[b], sem_row.at[b]).wait()

                def addrow(r, c2, _b=b):
                    for d in range(D // _L):
                        sl = pl.ds(d * _L, _L)
                        plsc.addupdate(ins[_b].at[r, sl], rows[_b][r, sl])
                    return c2

                lax.fori_loop(0, _C, addrow, 0, unroll=2)
                pltpu.async_copy(ins[b], out_hbm.at[pl.ds(obase + i * _C, _C)],
                                 sem_out.at[b])

            for b in range(_NBUF):
                i = (k + 1) * _NBUF + b

                @pl.when(i < chunks)
                def _(b=b, i=i):
                    prev = obase + (i - _NBUF) * _C
                    pltpu.make_async_copy(
                        ins[b], out_hbm.at[pl.ds(prev, _C)],
                        sem_out.at[b]).wait()
                    issue(i, b)

            return carry

        lax.fori_loop(0, nk, k_body, 0, unroll=False)

        for b in range(_NBUF):
            i = (nk - 1) * _NBUF + b
            pltpu.make_async_copy(
                ins[b], out_hbm.at[pl.ds(obase + i * _C, _C)],
                sem_out.at[b]).wait()

    return body(x, idx, table)


def _tc_body(x_ref, pos_ref, tab_ref, out_ref):
    pos = pos_ref[...]
    oh = (pos[:, None] == lax.broadcasted_iota(jnp.int32, (_G, 128), 1)
          ).astype(jnp.float32)
    gathered = jnp.dot(oh, tab_ref[...], preferred_element_type=jnp.float32)
    out_ref[...] = x_ref[...] + gathered


def _tc_body_aliased(x_ref, pos_ref, tab_ref, _sc_out_ref, out_ref):
    _tc_body(x_ref, pos_ref, tab_ref, out_ref)


def _tc_gather_add(x, idx, tab_pad, out_sc, m_tc):
    T, D = x.shape
    grid = (m_tc // _G,)
    return pl.pallas_call(
        _tc_body_aliased,
        grid=grid,
        in_specs=[
            pl.BlockSpec((_G, D), lambda i: (i, 0)),
            pl.BlockSpec((_G,), lambda i: (i,)),
            pl.BlockSpec((128, D), lambda i: (0, 0)),
            pl.BlockSpec(memory_space=pl.ANY),
        ],
        out_specs=pl.BlockSpec((_G, D), lambda i: (i, 0)),
        out_shape=jax.ShapeDtypeStruct((T, D), jnp.float32),
        input_output_aliases={3: 0},
    )(x, idx, tab_pad, out_sc)


def kernel(inputs, inputs_positions, position_emb):
    B, N, D = inputs.shape
    T = B * N
    x = inputs.reshape(T, D)
    idx = inputs_positions.reshape(T).astype(jnp.int32)
    table = position_emb.reshape(position_emb.shape[1], D)
    V = table.shape[0]

    m_sc = _M_SC
    m_tc = T - m_sc

    out_sc = _sc_gather_add(x, idx, table, m_tc, m_sc)
    tab_pad = jnp.pad(table, ((0, 128 - V), (0, 0)))
    out = _tc_gather_add(x, idx, tab_pad, out_sc, m_tc)
    return out.reshape(B, N, D)


# speedup vs baseline: 2.2146x; 1.1339x over previous
import functools

import jax
import jax.numpy as jnp
from jax import lax
from jax.experimental import pallas as pl
from jax.experimental.pallas import tpu as pltpu
from jax.experimental.pallas import tpu_sc as plsc

_NC = 2
_NS = 16
_NW = _NC * _NS
_L = 16
_C = 32
_NBUF = 4
_M_SC = 32768
_G = 2048


def _sc_gather_add(x, idx, table, sc_base, m_sc):
    T, D = x.shape
    R = m_sc // _NW
    chunks = R // _C
    nk = chunks // _NBUF

    mesh = plsc.VectorSubcoreMesh(core_axis_name="c", subcore_axis_name="s")

    buf = lambda: pltpu.VMEM((_C, D), jnp.float32)

    @functools.partial(
        pl.kernel,
        out_type=jax.ShapeDtypeStruct((T, D), jnp.float32),
        mesh=mesh,
        scratch_types=[
            pltpu.VMEM((R,), jnp.int32),
            buf(), buf(), buf(), buf(),
            buf(), buf(), buf(), buf(),
            pltpu.SemaphoreType.DMA((_NBUF,)),
            pltpu.SemaphoreType.DMA((_NBUF,)),
            pltpu.SemaphoreType.DMA((_NBUF,)),
        ],
    )
    def body(x_hbm, idx_hbm, tab_hbm, out_hbm, idx_all,
             in0, in1, in2, in3, row0, row1, row2, row3,
             sem_in, sem_row, sem_out):
        ins = (in0, in1, in2, in3)
        rows = (row0, row1, row2, row3)
        wid = lax.axis_index("s") * _NC + lax.axis_index("c")
        rbase = sc_base + wid * R
        obase = rbase

        pltpu.sync_copy(idx_hbm.at[pl.ds(rbase, R)], idx_all)

        def issue(i, b):
            pltpu.async_copy(x_hbm.at[pl.ds(rbase + i * _C, _C)],
                             ins[b], sem_in.at[b])
            pltpu.async_copy(tab_hbm.at[idx_all.at[pl.ds(i * _C, _C)]],
                             rows[b], sem_row.at[b])

        for b in range(_NBUF):
            issue(b, b)

        def k_body(k, carry):
            for b in range(_NBUF):
                i = k * _NBUF + b
                pltpu.make_async_copy(
                    x_hbm.at[pl.ds(rbase + i * _C, _C)],
                    ins[b], sem_in.at[b]).wait()
                pltpu.make_async_copy(
                    tab_hbm.at[idx_all.at[pl.ds(i * _C, _C)]],
                    rows[b], sem_row.at[b]).wait()

                def addrow(r, c2, _b=b):
                    for d in range(D // _L):
                        sl = pl.ds(d * _L, _L)
                        plsc.addupdate(ins[_b].at[r, sl], rows[_b][r, sl])
                    return c2

                lax.fori_loop(0, _C, addrow, 0, unroll=2)
                pltpu.async_copy(ins[b], out_hbm.at[pl.ds(obase + i * _C, _C)],
                                 sem_out.at[b])

            for b in range(_NBUF):
                i = (k + 1) * _NBUF + b

                @pl.when(i < chunks)
                def _(b=b, i=i):
                    prev = obase + (i - _NBUF) * _C
                    pltpu.make_async_copy(
                        ins[b], out_hbm.at[pl.ds(prev, _C)],
                        sem_out.at[b]).wait()
                    issue(i, b)

            return carry

        lax.fori_loop(0, nk, k_body, 0, unroll=False)

        for b in range(_NBUF):
            i = (nk - 1) * _NBUF + b
            pltpu.make_async_copy(
                ins[b], out_hbm.at[pl.ds(obase + i * _C, _C)],
                sem_out.at[b]).wait()

    return body(x, idx, table)


def _tc_body(x_ref, pos_ref, tab_ref, out_ref):
    pos = pos_ref[...]
    oh = (pos[:, None] == lax.broadcasted_iota(jnp.int32, (_G, 128), 1)
          ).astype(jnp.float32)
    gathered = jnp.dot(oh, tab_ref[...], preferred_element_type=jnp.float32)
    out_ref[...] = x_ref[...] + gathered


def _tc_body_aliased(x_ref, pos_ref, tab_ref, _sc_out_ref, out_ref):
    _tc_body(x_ref, pos_ref, tab_ref, out_ref)


def _tc_gather_add(x, idx, tab_pad, out_sc, m_tc):
    T, D = x.shape
    grid = (m_tc // _G,)
    return pl.pallas_call(
        _tc_body_aliased,
        grid=grid,
        in_specs=[
            pl.BlockSpec((_G, D), lambda i: (i, 0)),
            pl.BlockSpec((_G,), lambda i: (i,)),
            pl.BlockSpec((128, D), lambda i: (0, 0)),
            pl.BlockSpec(memory_space=pl.ANY),
        ],
        out_specs=pl.BlockSpec((_G, D), lambda i: (i, 0)),
        out_shape=jax.ShapeDtypeStruct((T, D), jnp.float32),
        input_output_aliases={3: 0},
    )(x, idx, tab_pad, out_sc)


def kernel(inputs, inputs_positions, position_emb):
    B, N, D = inputs.shape
    T = B * N
    x = inputs.reshape(T, D)
    idx = inputs_positions.reshape(T).astype(jnp.int32)
    table = position_emb.reshape(position_emb.shape[1], D)
    V = table.shape[0]

    m_sc = _M_SC
    m_tc = T - m_sc

    out_sc = _sc_gather_add(x, idx, table, m_tc, m_sc)
    tab_pad = jnp.pad(table, ((0, 128 - V), (0, 0)))
    out = _tc_gather_add(x, idx, tab_pad, out_sc, m_tc)
    return out.reshape(B, N, D)


# speedup vs baseline: 2.2719x; 1.0259x over previous
import functools

import jax
import jax.numpy as jnp
from jax import lax
from jax.experimental import pallas as pl
from jax.experimental.pallas import tpu as pltpu
from jax.experimental.pallas import tpu_sc as plsc

_NC = 2
_NS = 16
_NW = _NC * _NS
_L = 16
_C = 32
_NBUF = 4
_M_SC = 32768
_G = 4096


def _sc_gather_add(x, idx, table, sc_base, m_sc):
    T, D = x.shape
    R = m_sc // _NW
    chunks = R // _C
    nk = chunks // _NBUF

    mesh = plsc.VectorSubcoreMesh(core_axis_name="c", subcore_axis_name="s")

    buf = lambda: pltpu.VMEM((_C, D), jnp.float32)

    @functools.partial(
        pl.kernel,
        out_type=jax.ShapeDtypeStruct((T, D), jnp.float32),
        mesh=mesh,
        scratch_types=[
            pltpu.VMEM((R,), jnp.int32),
            buf(), buf(), buf(), buf(),
            buf(), buf(), buf(), buf(),
            pltpu.SemaphoreType.DMA((_NBUF,)),
            pltpu.SemaphoreType.DMA((_NBUF,)),
            pltpu.SemaphoreType.DMA((_NBUF,)),
        ],
    )
    def body(x_hbm, idx_hbm, tab_hbm, out_hbm, idx_all,
             in0, in1, in2, in3, row0, row1, row2, row3,
             sem_in, sem_row, sem_out):
        ins = (in0, in1, in2, in3)
        rows = (row0, row1, row2, row3)
        wid = lax.axis_index("s") * _NC + lax.axis_index("c")
        rbase = sc_base + wid * R
        obase = rbase

        pltpu.sync_copy(idx_hbm.at[pl.ds(rbase, R)], idx_all)

        def issue(i, b):
            pltpu.async_copy(x_hbm.at[pl.ds(rbase + i * _C, _C)],
                             ins[b], sem_in.at[b])
            pltpu.async_copy(tab_hbm.at[idx_all.at[pl.ds(i * _C, _C)]],
                             rows[b], sem_row.at[b])

        for b in range(_NBUF):
            issue(b, b)

        def k_body(k, carry):
            for b in range(_NBUF):
                i = k * _NBUF + b
                pltpu.make_async_copy(
                    x_hbm.at[pl.ds(rbase + i * _C, _C)],
                    ins[b], sem_in.at[b]).wait()
                pltpu.make_async_copy(
                    tab_hbm.at[idx_all.at[pl.ds(i * _C, _C)]],
                    rows[b], sem_row.at[b]).wait()

                def addrow(r, c2, _b=b):
                    for d in range(D // _L):
                        sl = pl.ds(d * _L, _L)
                        plsc.addupdate(ins[_b].at[r, sl], rows[_b][r, sl])
                    return c2

                lax.fori_loop(0, _C, addrow, 0, unroll=2)
                pltpu.async_copy(ins[b], out_hbm.at[pl.ds(obase + i * _C, _C)],
                                 sem_out.at[b])

            for b in range(_NBUF):
                i = (k + 1) * _NBUF + b

                @pl.when(i < chunks)
                def _(b=b, i=i):
                    prev = obase + (i - _NBUF) * _C
                    pltpu.make_async_copy(
                        ins[b], out_hbm.at[pl.ds(prev, _C)],
                        sem_out.at[b]).wait()
                    issue(i, b)

            return carry

        lax.fori_loop(0, nk, k_body, 0, unroll=False)

        for b in range(_NBUF):
            i = (nk - 1) * _NBUF + b
            pltpu.make_async_copy(
                ins[b], out_hbm.at[pl.ds(obase + i * _C, _C)],
                sem_out.at[b]).wait()

    return body(x, idx, table)


def _tc_body(x_ref, pos_ref, tab_ref, out_ref):
    pos = pos_ref[...]
    oh = (pos[:, None] == lax.broadcasted_iota(jnp.int32, (_G, 128), 1)
          ).astype(jnp.float32)
    gathered = jnp.dot(oh, tab_ref[...], preferred_element_type=jnp.float32)
    out_ref[...] = x_ref[...] + gathered


def _tc_body_aliased(x_ref, pos_ref, tab_ref, _sc_out_ref, out_ref):
    _tc_body(x_ref, pos_ref, tab_ref, out_ref)


def _tc_gather_add(x, idx, tab_pad, out_sc, m_tc):
    T, D = x.shape
    grid = (m_tc // _G,)
    return pl.pallas_call(
        _tc_body_aliased,
        grid=grid,
        in_specs=[
            pl.BlockSpec((_G, D), lambda i: (i, 0)),
            pl.BlockSpec((_G,), lambda i: (i,)),
            pl.BlockSpec((128, D), lambda i: (0, 0)),
            pl.BlockSpec(memory_space=pl.ANY),
        ],
        out_specs=pl.BlockSpec((_G, D), lambda i: (i, 0)),
        out_shape=jax.ShapeDtypeStruct((T, D), jnp.float32),
        input_output_aliases={3: 0},
    )(x, idx, tab_pad, out_sc)


def kernel(inputs, inputs_positions, position_emb):
    B, N, D = inputs.shape
    T = B * N
    x = inputs.reshape(T, D)
    idx = inputs_positions.reshape(T).astype(jnp.int32)
    table = position_emb.reshape(position_emb.shape[1], D)
    V = table.shape[0]

    m_sc = _M_SC
    m_tc = T - m_sc

    out_sc = _sc_gather_add(x, idx, table, m_tc, m_sc)
    tab_pad = jnp.pad(table, ((0, 128 - V), (0, 0)))
    out = _tc_gather_add(x, idx, tab_pad, out_sc, m_tc)
    return out.reshape(B, N, D)


# speedup vs baseline: 2.2816x; 1.0043x over previous
import functools

import jax
import jax.numpy as jnp
from jax import lax
from jax.experimental import pallas as pl
from jax.experimental.pallas import tpu as pltpu
from jax.experimental.pallas import tpu_sc as plsc

_NC = 2
_NS = 16
_NW = _NC * _NS
_L = 16
_C = 32
_NBUF = 4
_M_SC = 32768
_G = 8192


def _sc_gather_add(x, idx, table, sc_base, m_sc):
    T, D = x.shape
    R = m_sc // _NW
    chunks = R // _C
    nk = chunks // _NBUF

    mesh = plsc.VectorSubcoreMesh(core_axis_name="c", subcore_axis_name="s")

    buf = lambda: pltpu.VMEM((_C, D), jnp.float32)

    @functools.partial(
        pl.kernel,
        out_type=jax.ShapeDtypeStruct((T, D), jnp.float32),
        mesh=mesh,
        scratch_types=[
            pltpu.VMEM((R,), jnp.int32),
            buf(), buf(), buf(), buf(),
            buf(), buf(), buf(), buf(),
            pltpu.SemaphoreType.DMA((_NBUF,)),
            pltpu.SemaphoreType.DMA((_NBUF,)),
            pltpu.SemaphoreType.DMA((_NBUF,)),
        ],
    )
    def body(x_hbm, idx_hbm, tab_hbm, out_hbm, idx_all,
             in0, in1, in2, in3, row0, row1, row2, row3,
             sem_in, sem_row, sem_out):
        ins = (in0, in1, in2, in3)
        rows = (row0, row1, row2, row3)
        wid = lax.axis_index("s") * _NC + lax.axis_index("c")
        rbase = sc_base + wid * R
        obase = rbase

        pltpu.sync_copy(idx_hbm.at[pl.ds(rbase, R)], idx_all)

        def issue(i, b):
            pltpu.async_copy(x_hbm.at[pl.ds(rbase + i * _C, _C)],
                             ins[b], sem_in.at[b])
            pltpu.async_copy(tab_hbm.at[idx_all.at[pl.ds(i * _C, _C)]],
                             rows[b], sem_row.at[b])

        for b in range(_NBUF):
            issue(b, b)

        def k_body(k, carry):
            for b in range(_NBUF):
                i = k * _NBUF + b
                pltpu.make_async_copy(
                    x_hbm.at[pl.ds(rbase + i * _C, _C)],
                    ins[b], sem_in.at[b]).wait()
                pltpu.make_async_copy(
                    tab_hbm.at[idx_all.at[pl.ds(i * _C, _C)]],
                    rows[b], sem_row.at[b]).wait()

                def addrow(r, c2, _b=b):
                    for d in range(D // _L):
                        sl = pl.ds(d * _L, _L)
                        plsc.addupdate(ins[_b].at[r, sl], rows[_b][r, sl])
                    return c2

                lax.fori_loop(0, _C, addrow, 0, unroll=2)
                pltpu.async_copy(ins[b], out_hbm.at[pl.ds(obase + i * _C, _C)],
                                 sem_out.at[b])

            for b in range(_NBUF):
                i = (k + 1) * _NBUF + b

                @pl.when(i < chunks)
                def _(b=b, i=i):
                    prev = obase + (i - _NBUF) * _C
                    pltpu.make_async_copy(
                        ins[b], out_hbm.at[pl.ds(prev, _C)],
                        sem_out.at[b]).wait()
                    issue(i, b)

            return carry

        lax.fori_loop(0, nk, k_body, 0, unroll=False)

        for b in range(_NBUF):
            i = (nk - 1) * _NBUF + b
            pltpu.make_async_copy(
                ins[b], out_hbm.at[pl.ds(obase + i * _C, _C)],
                sem_out.at[b]).wait()

    return body(x, idx, table)


def _tc_body(x_ref, pos_ref, tab_ref, out_ref):
    pos = pos_ref[...]
    oh = (pos[:, None] == lax.broadcasted_iota(jnp.int32, (_G, 128), 1)
          ).astype(jnp.float32)
    gathered = jnp.dot(oh, tab_ref[...], preferred_element_type=jnp.float32)
    out_ref[...] = x_ref[...] + gathered


def _tc_body_aliased(x_ref, pos_ref, tab_ref, _sc_out_ref, out_ref):
    _tc_body(x_ref, pos_ref, tab_ref, out_ref)


def _tc_gather_add(x, idx, tab_pad, out_sc, m_tc):
    T, D = x.shape
    grid = (m_tc // _G,)
    return pl.pallas_call(
        _tc_body_aliased,
        grid=grid,
        in_specs=[
            pl.BlockSpec((_G, D), lambda i: (i, 0)),
            pl.BlockSpec((_G,), lambda i: (i,)),
            pl.BlockSpec((128, D), lambda i: (0, 0)),
            pl.BlockSpec(memory_space=pl.ANY),
        ],
        out_specs=pl.BlockSpec((_G, D), lambda i: (i, 0)),
        out_shape=jax.ShapeDtypeStruct((T, D), jnp.float32),
        input_output_aliases={3: 0},
    )(x, idx, tab_pad, out_sc)


def kernel(inputs, inputs_positions, position_emb):
    B, N, D = inputs.shape
    T = B * N
    x = inputs.reshape(T, D)
    idx = inputs_positions.reshape(T).astype(jnp.int32)
    table = position_emb.reshape(position_emb.shape[1], D)
    V = table.shape[0]

    m_sc = _M_SC
    m_tc = T - m_sc

    out_sc = _sc_gather_add(x, idx, table, m_tc, m_sc)
    tab_pad = jnp.pad(table, ((0, 128 - V), (0, 0)))
    out = _tc_gather_add(x, idx, tab_pad, out_sc, m_tc)
    return out.reshape(B, N, D)
